# baseline (speedup 1.0000x reference)
import functools

import jax
import jax.numpy as jnp
from jax import lax
from jax.experimental import pallas as pl
from jax.experimental.pallas import tpu as pltpu
from jax.experimental.pallas import tpu_sc as plsc

D = 1024
S = 4096
B = 4
NC, NS = 2, 16
NW = NC * NS
S_PER_W = S // NW
C = 32
W = C * D
L = 16


def _sc_add(x1d, pe1d):
    mesh = plsc.VectorSubcoreMesh(
        core_axis_name="c", subcore_axis_name="s", num_cores=NC, num_subcores=NS
    )

    @functools.partial(
        pl.kernel,
        out_type=jax.ShapeDtypeStruct((B * S * D,), jnp.float32),
        mesh=mesh,
        scratch_types=[
            pltpu.VMEM((W,), jnp.float32),
            pltpu.VMEM((W,), jnp.float32),
        ],
    )
    def k(x_hbm, pe_hbm, out_hbm, xbuf, pebuf):
        cid = lax.axis_index("c")
        sid = lax.axis_index("s")
        wid = sid * NC + cid
        s_base = wid * S_PER_W
        for g in range(S_PER_W // C):
            s0 = s_base + g * C
            pltpu.sync_copy(pe_hbm.at[pl.ds(s0 * D, W)], pebuf)
            for b in range(B):
                base = (b * S + s0) * D
                pltpu.sync_copy(x_hbm.at[pl.ds(base, W)], xbuf)

                def body(i):
                    plsc.addupdate(xbuf.at[pl.ds(i, L)], pebuf[pl.ds(i, L)])

                plsc.parallel_loop(0, W, L, unroll=8)(body)
                pltpu.sync_copy(xbuf, out_hbm.at[pl.ds(base, W)])

    return k(x1d, pe1d)


def kernel(x, pos_embedding):
    out = _sc_add(x.reshape(-1), pos_embedding.reshape(-1))
    return out.reshape(x.shape)


# speedup vs baseline: 1.1546x; 1.1546x over previous
import functools

import jax
import jax.numpy as jnp
from jax import lax
from jax.experimental import pallas as pl
from jax.experimental.pallas import tpu as pltpu
from jax.experimental.pallas import tpu_sc as plsc

D = 1024
S = 4096
B = 4
NC, NS = 2, 16
NW = NC * NS
S_PER_W = S // NW
C = 16
W = C * D
L = 16
NG = S_PER_W // C
CH = NG * B


def _sc_add(x1d, pe1d):
    mesh = plsc.VectorSubcoreMesh(
        core_axis_name="c", subcore_axis_name="s", num_cores=NC, num_subcores=NS
    )

    @functools.partial(
        pl.kernel,
        out_type=jax.ShapeDtypeStruct((B * S * D,), jnp.float32),
        mesh=mesh,
        scratch_types=[
            pltpu.VMEM((W,), jnp.float32),
            pltpu.VMEM((W,), jnp.float32),
            pltpu.VMEM((W,), jnp.float32),
            pltpu.VMEM((W,), jnp.float32),
            pltpu.SemaphoreType.DMA,
            pltpu.SemaphoreType.DMA,
            pltpu.SemaphoreType.DMA,
            pltpu.SemaphoreType.DMA,
            pltpu.SemaphoreType.DMA,
            pltpu.SemaphoreType.DMA,
        ],
    )
    def k(x_hbm, pe_hbm, out_hbm, xb0, xb1, pb0, pb1, sx0, sx1, so0, so1, sp0, sp1):
        xb, pb = (xb0, xb1), (pb0, pb1)
        sx, so, sp = (sx0, sx1), (so0, so1), (sp0, sp1)
        cid = lax.axis_index("c")
        sid = lax.axis_index("s")
        wid = sid * NC + cid
        s_base = wid * S_PER_W

        def x_off(kk):
            g, b = divmod(kk, B)
            return (b * S + s_base + g * C) * D

        def start_x(kk):
            return pltpu.async_copy(
                x_hbm.at[pl.ds(x_off(kk), W)], xb[kk % 2], sx[kk % 2]
            )

        def start_pe(g):
            return pltpu.async_copy(
                pe_hbm.at[pl.ds((s_base + g * C) * D, W)], pb[g % 2], sp[g % 2]
            )

        pe_d = [None, None]
        out_d = [None, None]
        pe_d[0] = start_pe(0)
        x_d = start_x(0)
        for kk in range(CH):
            p = kk % 2
            g, b = divmod(kk, B)
            if b == 0:
                pe_d[g % 2].wait()
            x_d.wait()
            if kk + 1 < CH:
                g1, b1 = divmod(kk + 1, B)
                if b1 == 0:
                    pe_d[g1 % 2] = start_pe(g1)
                if out_d[(kk + 1) % 2] is not None:
                    out_d[(kk + 1) % 2].wait()
                x_d = start_x(kk + 1)

            def body(i):
                plsc.addupdate(xb[p].at[pl.ds(i, L)], pb[g % 2][pl.ds(i, L)])

            plsc.parallel_loop(0, W, L, unroll=8)(body)
            out_d[p] = pltpu.async_copy(
                xb[p], out_hbm.at[pl.ds(x_off(kk), W)], so[p]
            )
        out_d[0].wait()
        out_d[1].wait()

    return k(x1d, pe1d)


def kernel(x, pos_embedding):
    out = _sc_add(x.reshape(-1), pos_embedding.reshape(-1))
    return out.reshape(x.shape)


# speedup vs baseline: 3.1920x; 2.7646x over previous
import functools

import jax
import jax.numpy as jnp
from jax import lax
from jax.experimental import pallas as pl
from jax.experimental.pallas import tpu as pltpu
from jax.experimental.pallas import tpu_sc as plsc

D = 1024
S = 4096
B = 4
NC, NS = 2, 16
NW = NC * NS
S_PER_W = S // NW
C = 16
L = 16
NG = S_PER_W // C
CH = NG * B


def _sc_add(x, pe):
    mesh = plsc.VectorSubcoreMesh(
        core_axis_name="c", subcore_axis_name="s", num_cores=NC, num_subcores=NS
    )

    @functools.partial(
        pl.kernel,
        out_type=jax.ShapeDtypeStruct((B, S, D), jnp.float32),
        mesh=mesh,
        scratch_types=[
            pltpu.VMEM((C, D), jnp.float32),
            pltpu.VMEM((C, D), jnp.float32),
            pltpu.VMEM((C, D), jnp.float32),
            pltpu.VMEM((C, D), jnp.float32),
            pltpu.SemaphoreType.DMA,
            pltpu.SemaphoreType.DMA,
            pltpu.SemaphoreType.DMA,
            pltpu.SemaphoreType.DMA,
            pltpu.SemaphoreType.DMA,
            pltpu.SemaphoreType.DMA,
        ],
    )
    def k(x_hbm, pe_hbm, out_hbm, xb0, xb1, pb0, pb1, sx0, sx1, so0, so1, sp0, sp1):
        xb, pb = (xb0, xb1), (pb0, pb1)
        sx, so, sp = (sx0, sx1), (so0, so1), (sp0, sp1)
        cid = lax.axis_index("c")
        sid = lax.axis_index("s")
        wid = sid * NC + cid
        s_base = wid * S_PER_W

        def start_x(kk):
            g, b = divmod(kk, B)
            return pltpu.async_copy(
                x_hbm.at[b, pl.ds(s_base + g * C, C), :], xb[kk % 2], sx[kk % 2]
            )

        def start_pe(g):
            return pltpu.async_copy(
                pe_hbm.at[pl.ds(s_base + g * C, C), :], pb[g % 2], sp[g % 2]
            )

        pe_d = [None, None]
        out_d = [None, None]
        pe_d[0] = start_pe(0)
        x_d = start_x(0)
        for kk in range(CH):
            p = kk % 2
            g, b = divmod(kk, B)
            if b == 0:
                pe_d[g % 2].wait()
            x_d.wait()
            if kk + 1 < CH:
                g1, b1 = divmod(kk + 1, B)
                if b1 == 0:
                    pe_d[g1 % 2] = start_pe(g1)
                if out_d[(kk + 1) % 2] is not None:
                    out_d[(kk + 1) % 2].wait()
                x_d = start_x(kk + 1)

            def body(i):
                r = lax.shift_right_logical(i, 10)
                c = pl.multiple_of(lax.bitwise_and(i, D - 1), L)
                plsc.addupdate(xb[p].at[r, pl.ds(c, L)], pb[g % 2][r, pl.ds(c, L)])

            plsc.parallel_loop(0, C * D, L, unroll=8)(body)
            out_d[p] = pltpu.async_copy(
                xb[p], out_hbm.at[kk % B, pl.ds(s_base + (kk // B) * C, C), :], so[p]
            )
        out_d[0].wait()
        out_d[1].wait()

    return k(x, pe)


def kernel(x, pos_embedding):
    return _sc_add(x, pos_embedding)
